# baseline (speedup 1.0000x reference)
import functools

import jax
import jax.numpy as jnp
from jax import lax
from jax.experimental import pallas as pl
from jax.experimental.pallas import tpu as pltpu
from jax.experimental.pallas import tpu_sc as plsc

N = 10000
D = 128
H = 128
E = 320000
SCALE = 20.0

NTILES = 32
C = 96
S = 106
SPAIRS = S // 2
EPAD = NTILES * S * C
NPAD = 10112
DUMMY = 10111
ROWS_Z = NPAD // 16

_PREC = jax.lax.Precision.DEFAULT


def _proj_body(x_ref, w_ref, b_ref, o_ref):
    o_ref[...] = (
        jnp.dot(x_ref[...], w_ref[...], preferred_element_type=jnp.float32,
                precision=_PREC)
        + b_ref[...]
    )


def _gin_mlp_body(h_ref, agg_ref, wa_ref, ba_ref, g_ref, be_ref, wb_ref,
                  bb_ref, o_ref, *, relu_out):
    h = h_ref[...]
    z = h + agg_ref[0, :N, :] + agg_ref[1, :N, :]
    z = jnp.dot(z, wa_ref[...], preferred_element_type=jnp.float32,
                precision=_PREC) + ba_ref[...]
    mu = jnp.mean(z, axis=0, keepdims=True)
    zc = z - mu
    var = jnp.mean(zc * zc, axis=0, keepdims=True)
    z = zc * lax.rsqrt(var + 1e-5) * g_ref[...] + be_ref[...]
    z = jnp.maximum(z, 0.0)
    z = jnp.dot(z, wb_ref[...], preferred_element_type=jnp.float32,
                precision=_PREC) + bb_ref[...]
    col_mean = jnp.mean(z, axis=0, keepdims=True)
    rownorm = jnp.sqrt(1e-6 + jnp.sum(z * z, axis=1, keepdims=True))
    z = SCALE * z / rownorm - col_mean
    if relu_out:
        z = jnp.maximum(z, 0.0)
    o_ref[...] = z


_proj = pl.pallas_call(
    _proj_body,
    out_shape=jax.ShapeDtypeStruct((N, H), jnp.float32),
)

_gin_mlp_relu = pl.pallas_call(
    functools.partial(_gin_mlp_body, relu_out=True),
    out_shape=jax.ShapeDtypeStruct((N, H), jnp.float32),
)

_gin_mlp_final = pl.pallas_call(
    functools.partial(_gin_mlp_body, relu_out=False),
    out_shape=jax.ShapeDtypeStruct((N, H), jnp.float32),
)


_sc_mesh = plsc.VectorSubcoreMesh(core_axis_name="c", subcore_axis_name="s")


@functools.partial(
    pl.kernel,
    out_type=jax.ShapeDtypeStruct((2, NPAD, H), jnp.float32),
    mesh=_sc_mesh,
    scratch_types=[
        pltpu.VMEM((S * C,), jnp.int32),
        pltpu.VMEM((S, C), jnp.int32),
        pltpu.VMEM((C, H), jnp.float32),
        pltpu.VMEM((C, H), jnp.float32),
        pltpu.VMEM_SHARED((NPAD, H), jnp.float32),
        pltpu.SemaphoreType.DMA,
        pltpu.SemaphoreType.DMA,
        pltpu.SemaphoreType.DMA,
        pltpu.SemaphoreType.DMA,
    ],
)
def _gin_agg(h_hbm, src_hbm, dst_hbm, zero_hbm, out_hbm,
             src_v, dst_v, r0, r1,
             agg, g0, g1, s0, s1):
    cid = lax.axis_index("c")
    sid = lax.axis_index("s")
    wid = sid * 2 + cid
    rows = (r0, r1)
    gsem = (g0, g1)
    ssem = (s0, s1)

    pltpu.sync_copy(src_hbm.at[wid], src_v)
    pltpu.sync_copy(dst_hbm.at[wid], dst_v)
    pltpu.sync_copy(zero_hbm.at[pl.ds(sid * ROWS_Z, ROWS_Z)],
                    agg.at[pl.ds(sid * ROWS_Z, ROWS_Z)])
    plsc.subcore_barrier()

    def src_at(j):
        return src_v.at[pl.ds(j * C, C)]

    for b in range(2):
        pltpu.async_copy(h_hbm.at[src_at(b)], rows[b], gsem[b])

    def pair(k, carry):
        for b in range(2):
            j = 2 * k + b
            pltpu.make_async_copy(h_hbm.at[src_at(j)], rows[b],
                                  gsem[b]).wait()
            pltpu.async_copy(rows[b], agg.at[dst_v.at[j]], ssem[b],
                             add=True)
        for b in range(2):
            j = 2 * k + b
            pltpu.make_async_copy(rows[b], agg.at[dst_v.at[j]],
                                  ssem[b]).wait()
            pltpu.async_copy(h_hbm.at[src_at((j + 2) % S)], rows[b],
                             gsem[b])
        return carry

    lax.fori_loop(0, SPAIRS, pair, 0)

    for b in range(2):
        pltpu.make_async_copy(h_hbm.at[src_at(b)], rows[b],
                              gsem[b]).wait()

    plsc.subcore_barrier()
    pltpu.sync_copy(agg.at[pl.ds(sid * ROWS_Z, ROWS_Z)],
                    out_hbm.at[cid, pl.ds(sid * ROWS_Z, ROWS_Z)])


def kernel(x, edge_index, W0, b0, W1a, b1a, g1, be1, W1b, b1b,
           W2a, b2a, g2, be2, W2b, b2b):
    src = edge_index[0]
    dst = edge_index[1]
    pad = EPAD - E
    src_p = jnp.concatenate(
        [src, jnp.zeros((pad,), jnp.int32)]).reshape(NTILES, S * C)
    dst_p = jnp.concatenate(
        [dst, jnp.full((pad,), DUMMY, jnp.int32)]).reshape(NTILES, S, C)
    zeros = jnp.zeros((NPAD, H), jnp.float32)

    b0r = b0.reshape(1, H)
    b1ar = b1a.reshape(1, H)
    g1r = g1.reshape(1, H)
    be1r = be1.reshape(1, H)
    b1br = b1b.reshape(1, H)
    b2ar = b2a.reshape(1, H)
    g2r = g2.reshape(1, H)
    be2r = be2.reshape(1, H)
    b2br = b2b.reshape(1, H)

    h = _proj(x, W0, b0r)
    agg1 = _gin_agg(h, src_p, dst_p, zeros)
    h = _gin_mlp_relu(h, agg1, W1a, b1ar, g1r, be1r, W1b, b1br)
    agg2 = _gin_agg(h, src_p, dst_p, zeros)
    h = _gin_mlp_final(h, agg2, W2a, b2ar, g2r, be2r, W2b, b2br)
    return h


# speedup vs baseline: 1.2430x; 1.2430x over previous
import functools

import jax
import jax.numpy as jnp
from jax import lax
from jax.experimental import pallas as pl
from jax.experimental.pallas import tpu as pltpu
from jax.experimental.pallas import tpu_sc as plsc

N = 10000
D = 128
H = 128
G = 64
E = 320000
SCALE = 20.0

NT = 16
C = 128
S = 160
NBUF = 4
EPAD = NT * S * C
NPAD = 10112
DUMMY = 10111
ROWS_Z = NPAD // 16

_PREC = jax.lax.Precision.DEFAULT


def _split_store(o_ref, z):
    o_ref[0] = z[:, :G]
    o_ref[1] = z[:, G:]


def _proj_body(x_ref, w_ref, b_ref, o_ref):
    z = (
        jnp.dot(x_ref[...], w_ref[...], preferred_element_type=jnp.float32,
                precision=_PREC)
        + b_ref[...]
    )
    _split_store(o_ref, z)


def _gin_mlp_body(h_ref, agg_ref, wa_ref, ba_ref, g_ref, be_ref, wb_ref,
                  bb_ref, o_ref, *, relu_out):
    h = jnp.concatenate([h_ref[0], h_ref[1]], axis=1)
    agg = jnp.concatenate([agg_ref[0, :N, :], agg_ref[1, :N, :]], axis=1)
    z = h + agg
    z = jnp.dot(z, wa_ref[...], preferred_element_type=jnp.float32,
                precision=_PREC) + ba_ref[...]
    mu = jnp.mean(z, axis=0, keepdims=True)
    zc = z - mu
    var = jnp.mean(zc * zc, axis=0, keepdims=True)
    z = zc * lax.rsqrt(var + 1e-5) * g_ref[...] + be_ref[...]
    z = jnp.maximum(z, 0.0)
    z = jnp.dot(z, wb_ref[...], preferred_element_type=jnp.float32,
                precision=_PREC) + bb_ref[...]
    col_mean = jnp.mean(z, axis=0, keepdims=True)
    rownorm = jnp.sqrt(1e-6 + jnp.sum(z * z, axis=1, keepdims=True))
    z = SCALE * z / rownorm - col_mean
    if relu_out:
        z = jnp.maximum(z, 0.0)
        _split_store(o_ref, z)
    else:
        o_ref[...] = z


_proj = pl.pallas_call(
    _proj_body,
    out_shape=jax.ShapeDtypeStruct((2, N, G), jnp.float32),
)

_gin_mlp_relu = pl.pallas_call(
    functools.partial(_gin_mlp_body, relu_out=True),
    out_shape=jax.ShapeDtypeStruct((2, N, G), jnp.float32),
)

_gin_mlp_final = pl.pallas_call(
    functools.partial(_gin_mlp_body, relu_out=False),
    out_shape=jax.ShapeDtypeStruct((N, H), jnp.float32),
)


_sc_mesh = plsc.VectorSubcoreMesh(core_axis_name="c", subcore_axis_name="s")


@functools.partial(
    pl.kernel,
    out_type=jax.ShapeDtypeStruct((2, NPAD, G), jnp.float32),
    mesh=_sc_mesh,
    compiler_params=pltpu.CompilerParams(use_tc_tiling_on_sc=False),
    scratch_types=[
        pltpu.VMEM((S * C,), jnp.int32),
        pltpu.VMEM((S, C), jnp.int32),
        pltpu.VMEM((C, G), jnp.float32),
        pltpu.VMEM((C, G), jnp.float32),
        pltpu.VMEM((C, G), jnp.float32),
        pltpu.VMEM((C, G), jnp.float32),
        pltpu.VMEM_SHARED((NPAD, G), jnp.float32),
        pltpu.SemaphoreType.DMA,
        pltpu.SemaphoreType.DMA,
        pltpu.SemaphoreType.DMA,
        pltpu.SemaphoreType.DMA,
        pltpu.SemaphoreType.DMA,
        pltpu.SemaphoreType.DMA,
        pltpu.SemaphoreType.DMA,
        pltpu.SemaphoreType.DMA,
    ],
)
def _gin_agg(h_hbm, src_hbm, dst_hbm, zero_hbm, out_hbm,
             src_v, dst_v, r0, r1, r2, r3,
             agg, g0, g1, g2, g3, s0, s1, s2, s3):
    cid = lax.axis_index("c")
    sid = lax.axis_index("s")
    rows = (r0, r1, r2, r3)
    gsem = (g0, g1, g2, g3)
    ssem = (s0, s1, s2, s3)
    hsrc = h_hbm.at[cid]

    pltpu.sync_copy(src_hbm.at[sid], src_v)
    pltpu.sync_copy(dst_hbm.at[sid], dst_v)
    pltpu.sync_copy(zero_hbm.at[pl.ds(sid * ROWS_Z, ROWS_Z)],
                    agg.at[pl.ds(sid * ROWS_Z, ROWS_Z)])
    plsc.subcore_barrier()

    def src_at(j):
        return src_v.at[pl.ds(j * C, C)]

    def wait_gather(j, b):
        pltpu.make_async_copy(hsrc.at[src_at(j)], rows[b], gsem[b]).wait()

    def issue_scatter(j, b):
        pltpu.async_copy(rows[b], agg.at[dst_v.at[j]], ssem[b], add=True)

    def wait_scatter(j, b):
        pltpu.make_async_copy(rows[b], agg.at[dst_v.at[j]], ssem[b]).wait()

    for j in range(2):
        pltpu.async_copy(hsrc.at[src_at(j)], rows[j], gsem[j])
    for j in range(2):
        wait_gather(j, j)
        issue_scatter(j, j)
        pltpu.async_copy(hsrc.at[src_at(j + 2)], rows[j + 2], gsem[j + 2])

    def step_body(j, b, nb, jnext):
        wait_gather(j, b)
        issue_scatter(j, b)
        wait_scatter(j - 2, nb)
        pltpu.async_copy(hsrc.at[src_at(jnext)], rows[nb], gsem[nb])

    def block(k, carry):
        j0 = 4 * k + 2
        for i in range(4):
            b = (2 + i) % NBUF
            step_body(j0 + i, b, (b + 2) % NBUF, j0 + i + 2)
        return carry

    lax.fori_loop(0, (S - 4) // 4, block, 0)

    step_body(S - 2, 2, 0, 0)
    step_body(S - 1, 3, 1, 1)

    for j in range(S - 2, S):
        wait_scatter(j, j % NBUF)
    for j in range(2):
        wait_gather(j, j)

    plsc.subcore_barrier()
    pltpu.sync_copy(agg.at[pl.ds(sid * ROWS_Z, ROWS_Z)],
                    out_hbm.at[cid, pl.ds(sid * ROWS_Z, ROWS_Z)])


def kernel(x, edge_index, W0, b0, W1a, b1a, g1, be1, W1b, b1b,
           W2a, b2a, g2, be2, W2b, b2b):
    src = edge_index[0]
    dst = edge_index[1]
    pad = EPAD - E
    src_p = jnp.concatenate(
        [src, jnp.zeros((pad,), jnp.int32)]).reshape(NT, S * C)
    dst_p = jnp.concatenate(
        [dst, jnp.full((pad,), DUMMY, jnp.int32)]).reshape(NT, S, C)
    zeros = jnp.zeros((NPAD, G), jnp.float32)

    b0r = b0.reshape(1, H)
    b1ar = b1a.reshape(1, H)
    g1r = g1.reshape(1, H)
    be1r = be1.reshape(1, H)
    b1br = b1b.reshape(1, H)
    b2ar = b2a.reshape(1, H)
    g2r = g2.reshape(1, H)
    be2r = be2.reshape(1, H)
    b2br = b2b.reshape(1, H)

    h = _proj(x, W0, b0r)
    agg1 = _gin_agg(h, src_p, dst_p, zeros)
    h = _gin_mlp_relu(h, agg1, W1a, b1ar, g1r, be1r, W1b, b1br)
    agg2 = _gin_agg(h, src_p, dst_p, zeros)
    h = _gin_mlp_final(h, agg2, W2a, b2ar, g2r, be2r, W2b, b2br)
    return h


# speedup vs baseline: 1.2707x; 1.0223x over previous
import functools

import jax
import jax.numpy as jnp
from jax import lax
from jax.experimental import pallas as pl
from jax.experimental.pallas import tpu as pltpu
from jax.experimental.pallas import tpu_sc as plsc

N = 10000
D = 128
H = 128
G = 64
E = 320000
SCALE = 20.0

NT = 16
C = 128
S = 160
NBUF = 4
EPAD = NT * S * C
NPAD = 10112
DUMMY = 10111
ROWS_Z = NPAD // 16

_PREC = jax.lax.Precision.DEFAULT


def _split_store(o_ref, z):
    o_ref[0] = z[:, :G]
    o_ref[1] = z[:, G:]


def _proj_body(x_ref, w_ref, b_ref, o_ref):
    z = (
        jnp.dot(x_ref[...], w_ref[...], preferred_element_type=jnp.float32,
                precision=_PREC)
        + b_ref[...]
    )
    _split_store(o_ref, z)


def _gin_mlp_body(h_ref, agg_ref, wa_ref, ba_ref, g_ref, be_ref, wb_ref,
                  bb_ref, o_ref, *, relu_out):
    h = jnp.concatenate([h_ref[0], h_ref[1]], axis=1)
    agg = jnp.concatenate([agg_ref[0, :N, :], agg_ref[1, :N, :]], axis=1)
    z = h + agg
    z = jnp.dot(z, wa_ref[...], preferred_element_type=jnp.float32,
                precision=_PREC) + ba_ref[...]
    mu = jnp.mean(z, axis=0, keepdims=True)
    zc = z - mu
    var = jnp.mean(zc * zc, axis=0, keepdims=True)
    z = zc * lax.rsqrt(var + 1e-5) * g_ref[...] + be_ref[...]
    z = jnp.maximum(z, 0.0)
    z = jnp.dot(z, wb_ref[...], preferred_element_type=jnp.float32,
                precision=_PREC) + bb_ref[...]
    col_mean = jnp.mean(z, axis=0, keepdims=True)
    rownorm = jnp.sqrt(1e-6 + jnp.sum(z * z, axis=1, keepdims=True))
    z = SCALE * z / rownorm - col_mean
    if relu_out:
        z = jnp.maximum(z, 0.0)
        _split_store(o_ref, z)
    else:
        o_ref[...] = z


_proj = pl.pallas_call(
    _proj_body,
    out_shape=jax.ShapeDtypeStruct((2, N, G), jnp.float32),
)

_gin_mlp_relu = pl.pallas_call(
    functools.partial(_gin_mlp_body, relu_out=True),
    out_shape=jax.ShapeDtypeStruct((2, N, G), jnp.float32),
)

_gin_mlp_final = pl.pallas_call(
    functools.partial(_gin_mlp_body, relu_out=False),
    out_shape=jax.ShapeDtypeStruct((N, H), jnp.float32),
)


_sc_mesh = plsc.VectorSubcoreMesh(core_axis_name="c", subcore_axis_name="s")


@functools.partial(
    pl.kernel,
    out_type=jax.ShapeDtypeStruct((2, NPAD, G), jnp.float32),
    mesh=_sc_mesh,
    compiler_params=pltpu.CompilerParams(use_tc_tiling_on_sc=False),
    scratch_types=[
        pltpu.VMEM((S * C,), jnp.int32),
        pltpu.VMEM((S, C), jnp.int32),
        pltpu.VMEM((C, G), jnp.float32),
        pltpu.VMEM((C, G), jnp.float32),
        pltpu.VMEM((C, G), jnp.float32),
        pltpu.VMEM((C, G), jnp.float32),
        pltpu.VMEM_SHARED((NPAD, G), jnp.float32),
        pltpu.SemaphoreType.DMA,
        pltpu.SemaphoreType.DMA,
        pltpu.SemaphoreType.DMA,
        pltpu.SemaphoreType.DMA,
        pltpu.SemaphoreType.DMA,
        pltpu.SemaphoreType.DMA,
        pltpu.SemaphoreType.DMA,
        pltpu.SemaphoreType.DMA,
    ],
)
def _gin_agg(h_hbm, src_hbm, dst_hbm, zero_hbm, out_hbm,
             src_v, dst_v, r0, r1, r2, r3,
             agg, g0, g1, g2, g3, s0, s1, s2, s3):
    cid = lax.axis_index("c")
    sid = lax.axis_index("s")
    rows = (r0, r1, r2, r3)
    gsem = (g0, g1, g2, g3)
    ssem = (s0, s1, s2, s3)
    hsrc = h_hbm.at[cid]

    pltpu.sync_copy(src_hbm.at[sid], src_v)
    pltpu.sync_copy(dst_hbm.at[sid], dst_v)
    pltpu.sync_copy(zero_hbm.at[pl.ds(sid * ROWS_Z, ROWS_Z)],
                    agg.at[pl.ds(sid * ROWS_Z, ROWS_Z)])
    plsc.subcore_barrier()

    def src_at(j):
        return src_v.at[pl.ds(j * C, C)]

    def wait_gather(j, b):
        pltpu.make_async_copy(hsrc.at[src_at(j)], rows[b], gsem[b]).wait()

    def issue_scatter(j, b):
        pltpu.async_copy(rows[b], agg.at[dst_v.at[j]], ssem[b], add=True)

    def wait_scatter(j, b):
        pltpu.make_async_copy(rows[b], agg.at[dst_v.at[j]], ssem[b]).wait()

    for j in range(3):
        pltpu.async_copy(hsrc.at[src_at(j)], rows[j], gsem[j])
    wait_gather(0, 0)
    issue_scatter(0, 0)
    pltpu.async_copy(hsrc.at[src_at(3)], rows[3], gsem[3])

    def step_body(j, b, nb, jnext):
        wait_gather(j, b)
        issue_scatter(j, b)
        wait_scatter(j - 1, nb)
        pltpu.async_copy(hsrc.at[src_at(jnext)], rows[nb], gsem[nb])

    def block(k, carry):
        j0 = 4 * k + 1
        for i in range(4):
            b = (1 + i) % NBUF
            step_body(j0 + i, b, (b + 3) % NBUF, j0 + i + 3)
        return carry

    lax.fori_loop(0, (S - 4) // 4, block, 0)

    step_body(S - 3, 1, 0, 0)
    step_body(S - 2, 2, 1, 1)
    step_body(S - 1, 3, 2, 2)

    wait_scatter(S - 1, 3)
    for j in range(3):
        wait_gather(j, j)

    plsc.subcore_barrier()
    pltpu.sync_copy(agg.at[pl.ds(sid * ROWS_Z, ROWS_Z)],
                    out_hbm.at[cid, pl.ds(sid * ROWS_Z, ROWS_Z)])


def kernel(x, edge_index, W0, b0, W1a, b1a, g1, be1, W1b, b1b,
           W2a, b2a, g2, be2, W2b, b2b):
    src = edge_index[0]
    dst = edge_index[1]
    pad = EPAD - E
    src_p = jnp.concatenate(
        [src, jnp.zeros((pad,), jnp.int32)]).reshape(NT, S * C)
    dst_p = jnp.concatenate(
        [dst, jnp.full((pad,), DUMMY, jnp.int32)]).reshape(NT, S, C)
    zeros = jnp.zeros((NPAD, G), jnp.float32)

    b0r = b0.reshape(1, H)
    b1ar = b1a.reshape(1, H)
    g1r = g1.reshape(1, H)
    be1r = be1.reshape(1, H)
    b1br = b1b.reshape(1, H)
    b2ar = b2a.reshape(1, H)
    g2r = g2.reshape(1, H)
    be2r = be2.reshape(1, H)
    b2br = b2b.reshape(1, H)

    h = _proj(x, W0, b0r)
    agg1 = _gin_agg(h, src_p, dst_p, zeros)
    h = _gin_mlp_relu(h, agg1, W1a, b1ar, g1r, be1r, W1b, b1br)
    agg2 = _gin_agg(h, src_p, dst_p, zeros)
    h = _gin_mlp_final(h, agg2, W2a, b2ar, g2r, be2r, W2b, b2br)
    return h


# speedup vs baseline: 1.3044x; 1.0265x over previous
import functools

import jax
import jax.numpy as jnp
from jax import lax
from jax.experimental import pallas as pl
from jax.experimental.pallas import tpu as pltpu
from jax.experimental.pallas import tpu_sc as plsc

N = 10000
D = 128
H = 128
G = 64
E = 320000
SCALE = 20.0

NT = 16
C = 128
S = 160
NBUF = 6
EPAD = NT * S * C
NPAD = 10112
DUMMY = 10111
ROWS_Z = NPAD // 16

_PREC = jax.lax.Precision.DEFAULT


def _split_store(o_ref, z):
    o_ref[0] = z[:, :G]
    o_ref[1] = z[:, G:]


def _proj_body(x_ref, w_ref, b_ref, o_ref):
    z = (
        jnp.dot(x_ref[...], w_ref[...], preferred_element_type=jnp.float32,
                precision=_PREC)
        + b_ref[...]
    )
    _split_store(o_ref, z)


def _gin_mlp_body(h_ref, agg_ref, wa_ref, ba_ref, g_ref, be_ref, wb_ref,
                  bb_ref, o_ref, *, relu_out):
    h = jnp.concatenate([h_ref[0], h_ref[1]], axis=1)
    agg = jnp.concatenate([agg_ref[0, :N, :], agg_ref[1, :N, :]], axis=1)
    z = h + agg
    z = jnp.dot(z, wa_ref[...], preferred_element_type=jnp.float32,
                precision=_PREC) + ba_ref[...]
    mu = jnp.mean(z, axis=0, keepdims=True)
    zc = z - mu
    var = jnp.mean(zc * zc, axis=0, keepdims=True)
    z = zc * lax.rsqrt(var + 1e-5) * g_ref[...] + be_ref[...]
    z = jnp.maximum(z, 0.0)
    z = jnp.dot(z, wb_ref[...], preferred_element_type=jnp.float32,
                precision=_PREC) + bb_ref[...]
    col_mean = jnp.mean(z, axis=0, keepdims=True)
    rownorm = jnp.sqrt(1e-6 + jnp.sum(z * z, axis=1, keepdims=True))
    z = SCALE * z / rownorm - col_mean
    if relu_out:
        z = jnp.maximum(z, 0.0)
        _split_store(o_ref, z)
    else:
        o_ref[...] = z


_proj = pl.pallas_call(
    _proj_body,
    out_shape=jax.ShapeDtypeStruct((2, N, G), jnp.float32),
)

_gin_mlp_relu = pl.pallas_call(
    functools.partial(_gin_mlp_body, relu_out=True),
    out_shape=jax.ShapeDtypeStruct((2, N, G), jnp.float32),
)

_gin_mlp_final = pl.pallas_call(
    functools.partial(_gin_mlp_body, relu_out=False),
    out_shape=jax.ShapeDtypeStruct((N, H), jnp.float32),
)


_sc_mesh = plsc.VectorSubcoreMesh(core_axis_name="c", subcore_axis_name="s")


@functools.partial(
    pl.kernel,
    out_type=jax.ShapeDtypeStruct((2, NPAD, G), jnp.float32),
    mesh=_sc_mesh,
    compiler_params=pltpu.CompilerParams(use_tc_tiling_on_sc=False),
    scratch_types=[
        pltpu.VMEM((S * C,), jnp.int32),
        pltpu.VMEM((S, C), jnp.int32),
        pltpu.VMEM((C, G), jnp.float32),
        pltpu.VMEM((C, G), jnp.float32),
        pltpu.VMEM((C, G), jnp.float32),
        pltpu.VMEM((C, G), jnp.float32),
        pltpu.VMEM((C, G), jnp.float32),
        pltpu.VMEM((C, G), jnp.float32),
        pltpu.VMEM_SHARED((NPAD, G), jnp.float32),
        pltpu.SemaphoreType.DMA,
        pltpu.SemaphoreType.DMA,
        pltpu.SemaphoreType.DMA,
        pltpu.SemaphoreType.DMA,
        pltpu.SemaphoreType.DMA,
        pltpu.SemaphoreType.DMA,
        pltpu.SemaphoreType.DMA,
        pltpu.SemaphoreType.DMA,
        pltpu.SemaphoreType.DMA,
        pltpu.SemaphoreType.DMA,
        pltpu.SemaphoreType.DMA,
        pltpu.SemaphoreType.DMA,
    ],
)
def _gin_agg(h_hbm, src_hbm, dst_hbm, zero_hbm, out_hbm,
             src_v, dst_v, r0, r1, r2, r3, r4, r5,
             agg, g0, g1, g2, g3, g4, g5, s0, s1, s2, s3, s4, s5):
    cid = lax.axis_index("c")
    sid = lax.axis_index("s")
    rows = (r0, r1, r2, r3, r4, r5)
    gsem = (g0, g1, g2, g3, g4, g5)
    ssem = (s0, s1, s2, s3, s4, s5)
    hsrc = h_hbm.at[cid]

    pltpu.sync_copy(src_hbm.at[sid], src_v)
    pltpu.sync_copy(dst_hbm.at[sid], dst_v)
    pltpu.sync_copy(zero_hbm.at[pl.ds(sid * ROWS_Z, ROWS_Z)],
                    agg.at[pl.ds(sid * ROWS_Z, ROWS_Z)])
    plsc.subcore_barrier()

    def src_at(j):
        return src_v.at[pl.ds(j * C, C)]

    def wait_gather(j, b):
        pltpu.make_async_copy(hsrc.at[src_at(j)], rows[b], gsem[b]).wait()

    def issue_scatter(j, b):
        pltpu.async_copy(rows[b], agg.at[dst_v.at[j]], ssem[b], add=True)

    def wait_scatter(j, b):
        pltpu.make_async_copy(rows[b], agg.at[dst_v.at[j]], ssem[b]).wait()

    for j in range(3):
        pltpu.async_copy(hsrc.at[src_at(j)], rows[j], gsem[j])
    for j in range(3):
        wait_gather(j, j)
        issue_scatter(j, j)
        pltpu.async_copy(hsrc.at[src_at(j + 3)], rows[j + 3], gsem[j + 3])

    def step_body(j, b, nb, jnext):
        wait_gather(j, b)
        issue_scatter(j, b)
        wait_scatter(j - 3, nb)
        pltpu.async_copy(hsrc.at[src_at(jnext)], rows[nb], gsem[nb])

    def block(k, carry):
        j0 = 6 * k + 3
        for i in range(6):
            j = j0 + i
            b = (3 + i) % NBUF
            step_body(j, b, (b + 3) % NBUF, lax.rem(j + 3, S))
        return carry

    lax.fori_loop(0, (S - 4) // 6, block, 0)

    step_body(S - 1, 3, 0, 2)

    wait_scatter(S - 3, 1)
    wait_scatter(S - 2, 2)
    wait_scatter(S - 1, 3)
    wait_gather(0, 4)
    wait_gather(1, 5)
    wait_gather(2, 0)

    plsc.subcore_barrier()
    pltpu.sync_copy(agg.at[pl.ds(sid * ROWS_Z, ROWS_Z)],
                    out_hbm.at[cid, pl.ds(sid * ROWS_Z, ROWS_Z)])


def kernel(x, edge_index, W0, b0, W1a, b1a, g1, be1, W1b, b1b,
           W2a, b2a, g2, be2, W2b, b2b):
    src = edge_index[0]
    dst = edge_index[1]
    pad = EPAD - E
    src_p = jnp.concatenate(
        [src, jnp.zeros((pad,), jnp.int32)]).reshape(NT, S * C)
    dst_p = jnp.concatenate(
        [dst, jnp.full((pad,), DUMMY, jnp.int32)]).reshape(NT, S, C)
    zeros = jnp.zeros((NPAD, G), jnp.float32)

    b0r = b0.reshape(1, H)
    b1ar = b1a.reshape(1, H)
    g1r = g1.reshape(1, H)
    be1r = be1.reshape(1, H)
    b1br = b1b.reshape(1, H)
    b2ar = b2a.reshape(1, H)
    g2r = g2.reshape(1, H)
    be2r = be2.reshape(1, H)
    b2br = b2b.reshape(1, H)

    h = _proj(x, W0, b0r)
    agg1 = _gin_agg(h, src_p, dst_p, zeros)
    h = _gin_mlp_relu(h, agg1, W1a, b1ar, g1r, be1r, W1b, b1br)
    agg2 = _gin_agg(h, src_p, dst_p, zeros)
    h = _gin_mlp_final(h, agg2, W2a, b2ar, g2r, be2r, W2b, b2br)
    return h
